# baseline (speedup 1.0000x reference)
import jax
import jax.numpy as jnp
from jax.experimental import pallas as pl

N = 16384
NUM_CLASSES = 1000
BLOCK_ROWS = 512
GRID = N // BLOCK_ROWS


def _onehot_block(x_ref, out_ref):
    idx = x_ref[0, 0, :].reshape(BLOCK_ROWS, 1)
    cols = jax.lax.broadcasted_iota(jnp.int32, (BLOCK_ROWS, NUM_CLASSES), 1)
    out_ref[...] = (idx == cols).astype(jnp.int32)


def kernel(x):
    x3 = x.reshape(GRID, 1, BLOCK_ROWS)
    return pl.pallas_call(
        _onehot_block,
        grid=(GRID,),
        in_specs=[pl.BlockSpec((1, 1, BLOCK_ROWS), lambda i: (i, 0, 0))],
        out_specs=pl.BlockSpec((BLOCK_ROWS, NUM_CLASSES), lambda i: (i, 0)),
        out_shape=jax.ShapeDtypeStruct((N, NUM_CLASSES), jnp.int32),
    )(x3)


# speedup vs baseline: 1.0751x; 1.0751x over previous
import jax
import jax.numpy as jnp
from jax.experimental import pallas as pl
from jax.experimental.pallas import tpu as pltpu

N = 16384
NUM_CLASSES = 1000
BLOCK_ROWS = 512
GRID = N // BLOCK_ROWS
NBUF = 8


def _onehot_block(x_ref, out_ref, scratch_ref, sems):
    i = pl.program_id(0)
    slot = jax.lax.rem(i, NBUF)

    @pl.when(i >= NBUF)
    def _wait_slot():
        pltpu.make_async_copy(
            scratch_ref.at[slot],
            out_ref.at[pl.ds((i - NBUF) * BLOCK_ROWS, BLOCK_ROWS), :],
            sems.at[slot],
        ).wait()

    idx = x_ref[0, 0, :].reshape(BLOCK_ROWS, 1)
    cols = jax.lax.broadcasted_iota(jnp.int32, (BLOCK_ROWS, NUM_CLASSES), 1)
    scratch_ref[slot] = (idx == cols).astype(jnp.int32)

    pltpu.make_async_copy(
        scratch_ref.at[slot],
        out_ref.at[pl.ds(i * BLOCK_ROWS, BLOCK_ROWS), :],
        sems.at[slot],
    ).start()

    @pl.when(i == GRID - 1)
    def _drain():
        for j in range(NBUF):
            step = GRID - NBUF + j
            s = step % NBUF
            pltpu.make_async_copy(
                scratch_ref.at[s],
                out_ref.at[pl.ds(step * BLOCK_ROWS, BLOCK_ROWS), :],
                sems.at[s],
            ).wait()


def kernel(x):
    x3 = x.reshape(GRID, 1, BLOCK_ROWS)
    return pl.pallas_call(
        _onehot_block,
        grid=(GRID,),
        in_specs=[pl.BlockSpec((1, 1, BLOCK_ROWS), lambda i: (i, 0, 0))],
        out_specs=pl.BlockSpec(memory_space=pl.ANY),
        out_shape=jax.ShapeDtypeStruct((N, NUM_CLASSES), jnp.int32),
        scratch_shapes=[
            pltpu.VMEM((NBUF, BLOCK_ROWS, NUM_CLASSES), jnp.int32),
            pltpu.SemaphoreType.DMA((NBUF,)),
        ],
    )(x3)
